# baseline (speedup 1.0000x reference)
import functools

import jax
import jax.numpy as jnp
from jax import lax
from jax.experimental import pallas as pl
from jax.experimental.pallas import tpu as pltpu
from jax.experimental.pallas import tpu_sc as plsc

N = 10000
E = 320000
D = 128
O = 64
ACC_ROWS = 10112
EROWS = 2560
CW = 128
NB = 2

_MESH = dict(core_axis_name="c", subcore_axis_name="s", num_cores=2,
             num_subcores=16)


def _zero_buf(rows):
    z = jnp.zeros((16,), jnp.float32)

    def body(r, carry):
        for q in range(8):
            rows[0, r, pl.ds(q * 16, 16)] = z
        return carry

    lax.fori_loop(0, 128, body, 0)


def _zero_acc_stripe(rows, acc, s):
    for t in range(4):
        pltpu.sync_copy(rows.at[0], acc.at[pl.ds(s * 632 + t * 128, 128)])
    pltpu.sync_copy(rows.at[0, pl.ds(0, 120)],
                    acc.at[pl.ds(s * 632 + 512, 120)])


def _copy_out(acc, out_hbm, c, s):
    pltpu.sync_copy(acc.at[pl.ds(s * 624, 624)],
                    out_hbm.at[c, pl.ds(s * 624, 624)])

    @pl.when(s == 15)
    def _():
        pltpu.sync_copy(acc.at[pl.ds(9984, 16)],
                        out_hbm.at[c, pl.ds(9984, 16)])


def _scale_rows(rows, b, wbuf, slot, wrow):
    def grp(g, carry):
        w16 = wbuf[slot, wrow, pl.ds(g * 16, 16)]
        for i in range(16):
            r = g * 16 + i
            wb = jnp.broadcast_to(w16[i], (16,))
            for q in range(8):
                sl = pl.ds(q * 16, 16)
                rows[b, r, sl] = rows[b, r, sl] * wb
        return carry

    lax.fori_loop(0, 8, grp, 0)


def _edge_loop(x_hbm, stage_idx_fn, src_idx, dst_idx, rows, acc,
               sem_g, base, n_chunks, ig, scale_fn):
    def g_slot(k):
        return ((k // ig) % 2, k % ig)

    stage_idx_fn(0)
    for b in range(NB):
        pltpu.async_copy(x_hbm.at[src_idx.at[g_slot(b)]], rows.at[b],
                         sem_g.at[b])

    def outer(jo, carry):
        for b in range(NB):
            k = jo * NB + b
            slot, row = g_slot(k)
            pltpu.make_async_copy(
                x_hbm.at[src_idx.at[slot, row]], rows.at[b],
                sem_g.at[b]).wait()
            scale_fn(rows, b, slot, row)
            pltpu.sync_copy(rows.at[b], acc.at[dst_idx.at[slot, row]],
                            add=True)

            @pl.when(jnp.logical_and((k + 2) % ig == 0, k + 2 < n_chunks))
            def _():
                stage_idx_fn((k + 2) // ig)

            @pl.when(k + 2 < n_chunks)
            def _():
                slot2, row2 = g_slot(k + 2)
                pltpu.async_copy(x_hbm.at[src_idx.at[slot2, row2]],
                                 rows.at[b], sem_g.at[b])
        return carry

    lax.fori_loop(0, n_chunks // NB, outer, 0)


def _conv1_loop(x_hbm, stage_idx_fn, src_idx, dst64, wbuf, rows, acc,
                sem_g, sem_s, base, n_chunks, ig):
    def g_slot(k):
        return ((k // ig) % 2, k % ig)

    stage_idx_fn(0)
    for b in range(NB):
        pltpu.async_copy(x_hbm.at[src_idx.at[g_slot(b)]], rows.at[b],
                         sem_g.at[b])

    def half(rows_, b, slot, row, h):
        def grp(g, carry):
            w16 = wbuf[slot, row, pl.ds(h * 64 + g * 16, 16)]
            for i in range(16):
                r = h * 64 + g * 16 + i
                wb = jnp.broadcast_to(w16[i], (16,))
                for q in range(8):
                    sl = pl.ds(q * 16, 16)
                    rows_[b, r, sl] = rows_[b, r, sl] * wb
            return carry

        lax.fori_loop(0, 4, grp, 0)

    def outer(jo, carry):
        for b in range(NB):
            k = jo * NB + b
            slot, row = g_slot(k)
            pltpu.make_async_copy(
                x_hbm.at[src_idx.at[slot, row]], rows.at[b],
                sem_g.at[b]).wait()
            half(rows, b, slot, row, 0)
            cpA = pltpu.async_copy(rows.at[b, pl.ds(0, 64)],
                                   acc.at[dst64.at[slot, 2 * row]],
                                   sem_s, add=True)
            half(rows, b, slot, row, 1)
            cpA.wait()
            pltpu.sync_copy(rows.at[b, pl.ds(64, 64)],
                            acc.at[dst64.at[slot, 2 * row + 1]], add=True)

            @pl.when(jnp.logical_and((k + 2) % ig == 0, k + 2 < n_chunks))
            def _():
                stage_idx_fn((k + 2) // ig)

            @pl.when(k + 2 < n_chunks)
            def _():
                slot2, row2 = g_slot(k + 2)
                pltpu.async_copy(x_hbm.at[src_idx.at[slot2, row2]],
                                 rows.at[b], sem_g.at[b])
        return carry

    lax.fori_loop(0, n_chunks // NB, outer, 0)


IG1 = 16
IG23 = 32


@functools.partial(
    pl.kernel,
    out_type=jax.ShapeDtypeStruct((2, N, D), jnp.float32),
    mesh=plsc.VectorSubcoreMesh(**_MESH),
    compiler_params=pltpu.CompilerParams(needs_layout_passes=False),
    scratch_types=[
        pltpu.VMEM((2, IG1, CW), jnp.int32),
        pltpu.VMEM((2, 2 * IG1, CW // 2), jnp.int32),
        pltpu.VMEM((2, IG1, CW), jnp.float32),
        pltpu.VMEM((NB, CW, D), jnp.float32),
        pltpu.VMEM_SHARED((ACC_ROWS, D), jnp.float32),
        pltpu.SemaphoreType.DMA((NB,)),
        pltpu.SemaphoreType.DMA,
    ],
)
def _sc_conv1(x_hbm, eix_hbm, eix64_hbm, w_hbm, out_hbm,
              src_idx, dst64, wbuf, rows, acc, sem_g, sem_s):
    c = lax.axis_index("c")
    s = lax.axis_index("s")
    base = (c * 16 + s) * 80
    n_chunks = 80

    _zero_buf(rows)
    _zero_acc_stripe(rows, acc, s)
    plsc.subcore_barrier()

    def stage_idx_fn(g):
        rb = base + g * IG1
        slot = g % 2
        pltpu.sync_copy(eix_hbm.at[0, pl.ds(rb, IG1)], src_idx.at[slot])
        pltpu.sync_copy(eix64_hbm.at[1, pl.ds(2 * rb, 2 * IG1)],
                        dst64.at[slot])
        pltpu.sync_copy(w_hbm.at[pl.ds(rb, IG1)], wbuf.at[slot])

    _conv1_loop(x_hbm, stage_idx_fn, src_idx, dst64, wbuf, rows, acc,
                sem_g, sem_s, base, n_chunks, IG1)

    plsc.subcore_barrier()
    _copy_out(acc, out_hbm, c, s)


@functools.partial(
    pl.kernel,
    out_type=jax.ShapeDtypeStruct((2, N, D), jnp.float32),
    mesh=plsc.VectorSubcoreMesh(**_MESH),
    compiler_params=pltpu.CompilerParams(needs_layout_passes=False),
    scratch_types=[
        pltpu.VMEM((2, IG23, CW), jnp.int32),
        pltpu.VMEM((2, IG23, CW), jnp.int32),
        pltpu.VMEM((NB, CW, D), jnp.float32),
        pltpu.VMEM_SHARED((ACC_ROWS, D), jnp.float32),
        pltpu.SemaphoreType.DMA((NB,)),
    ],
)
def _sc_conv23(x2_hbm, x3_hbm, eix_hbm, out_hbm,
               src_idx, dst_idx, rows, acc, sem_g):
    c = lax.axis_index("c")
    s = lax.axis_index("s")
    base = s * 160
    n_chunks = 160

    _zero_buf(rows)
    _zero_acc_stripe(rows, acc, s)
    plsc.subcore_barrier()

    def stage_idx_fn(g):
        rb = base + g * IG23
        slot = g % 2
        pltpu.sync_copy(eix_hbm.at[0, pl.ds(rb, IG23)], src_idx.at[slot])
        pltpu.sync_copy(eix_hbm.at[1, pl.ds(rb, IG23)], dst_idx.at[slot])

    noscale = lambda rows_, b, slot, row: None

    @pl.when(c == 0)
    def _():
        _edge_loop(x2_hbm, stage_idx_fn, src_idx, dst_idx, rows,
                   acc, sem_g, base, n_chunks, IG23, noscale)

    @pl.when(c == 1)
    def _():
        _edge_loop(x3_hbm, stage_idx_fn, src_idx, dst_idx, rows,
                   acc, sem_g, base, n_chunks, IG23, noscale)

    plsc.subcore_barrier()
    _copy_out(acc, out_hbm, c, s)


def _tc_conv1_combine(p1, x_meas, W_rel1, b_rel1, W_root1):
    BR = 2000
    grid = (N // BR,)

    def body(p1_ref, xm_ref, wr1_ref, b1_ref, wo1_ref, mov_ref):
        f32 = jnp.float32
        a1 = p1_ref[0] + p1_ref[1]
        m = (jnp.dot(a1, wr1_ref[...], preferred_element_type=f32)
             + b1_ref[...]
             + jnp.dot(xm_ref[...], wo1_ref[...], preferred_element_type=f32))
        mov_ref[...] = jnp.maximum(m, 0.0)

    full = lambda shape: pl.BlockSpec(shape, lambda i: (0,) * len(shape))
    return pl.pallas_call(
        body,
        grid=grid,
        in_specs=[
            pl.BlockSpec((2, BR, D), lambda i: (0, i, 0)),
            pl.BlockSpec((BR, D), lambda i: (i, 0)),
            full((D, D)), full((1, D)), full((D, D)),
        ],
        out_specs=pl.BlockSpec((BR, D), lambda i: (i, 0)),
        out_shape=jax.ShapeDtypeStruct((N, D), jnp.float32),
    )(p1, x_meas, W_rel1, b_rel1.reshape(1, D), W_root1)


def _tc_final(agg23, x_dem, W_rel2, b_rel2, W_root2,
              W_rel3, b_rel3, W_root3, W_lin, b_lin):
    BR = 2000
    grid = (N // BR,)

    def body(agg_ref, xd_ref, wr2_ref, b2_ref, wo2_ref,
             wr3_ref, b3_ref, wo3_ref, wl_ref, bl_ref, out_ref):
        f32 = jnp.float32
        a2 = agg_ref[0]
        a3 = agg_ref[1]
        u1 = (jnp.dot(a2, wr2_ref[...], preferred_element_type=f32)
              + b2_ref[...]
              + jnp.dot(xd_ref[...], wo2_ref[...], preferred_element_type=f32))
        u1 = jnp.maximum(u1, 0.0)
        u = (jnp.dot(a3, wr3_ref[...], preferred_element_type=f32)
             + b3_ref[...]
             + jnp.dot(u1, wo3_ref[...], preferred_element_type=f32))
        u = jnp.maximum(u, 0.0)
        out_ref[...] = (jnp.dot(u, wl_ref[...], preferred_element_type=f32)
                        + bl_ref[...])

    full = lambda shape: pl.BlockSpec(shape, lambda i: (0,) * len(shape))
    return pl.pallas_call(
        body,
        grid=grid,
        in_specs=[
            pl.BlockSpec((2, BR, D), lambda i: (0, i, 0)),
            pl.BlockSpec((BR, D), lambda i: (i, 0)),
            full((D, D)), full((1, D)), full((D, D)),
            full((D, D)), full((1, D)), full((D, D)),
            full((D, O)), full((1, O)),
        ],
        out_specs=pl.BlockSpec((BR, O), lambda i: (i, 0)),
        out_shape=jax.ShapeDtypeStruct((N, O), jnp.float32),
    )(agg23, x_dem, W_rel2, b_rel2.reshape(1, D), W_root2,
      W_rel3, b_rel3.reshape(1, D), W_root3, W_lin, b_lin.reshape(1, O))


def _pad_eix(eix):
    pe = EROWS * CW - E
    ar = jnp.arange(pe, dtype=jnp.int32)
    pad = jnp.stack([ar % N, N + ar % (ACC_ROWS - N)])
    return jnp.concatenate([eix, pad], axis=1).reshape(2, EROWS, CW)


def kernel(x_measurement, x_demand, edge_index_mp, edge_index_rev,
           edge_weight, W_rel1, b_rel1, W_root1, W_rel2, b_rel2, W_root2,
           W_rel3, b_rel3, W_root3, W_lin, b_lin):
    eix_mp = _pad_eix(edge_index_mp)
    eix_mp64 = eix_mp.reshape(2, 2 * EROWS, CW // 2)
    eix_rv = _pad_eix(edge_index_rev)
    w_mp = jnp.pad(edge_weight, (0, EROWS * CW - E)).reshape(EROWS, CW)

    p1 = _sc_conv1(x_measurement, eix_mp, eix_mp64, w_mp)
    movie_x = _tc_conv1_combine(p1, x_measurement, W_rel1, b_rel1, W_root1)
    agg23 = _sc_conv23(x_measurement, movie_x, eix_rv)
    return _tc_final(agg23, x_demand, W_rel2, b_rel2, W_root2,
                     W_rel3, b_rel3, W_root3, W_lin, b_lin)


# speedup vs baseline: 1.0063x; 1.0063x over previous
import functools

import jax
import jax.numpy as jnp
from jax import lax
from jax.experimental import pallas as pl
from jax.experimental.pallas import tpu as pltpu
from jax.experimental.pallas import tpu_sc as plsc

N = 10000
E = 320000
D = 128
O = 64
ACC_ROWS = 10112
EROWS = 2560
CW = 128
NB = 2

_MESH = dict(core_axis_name="c", subcore_axis_name="s", num_cores=2,
             num_subcores=16)


def _zero_buf(rows):
    z = jnp.zeros((16,), jnp.float32)

    def body(r, carry):
        for q in range(8):
            rows[0, r, pl.ds(q * 16, 16)] = z
        return carry

    lax.fori_loop(0, 128, body, 0)


def _zero_acc_stripe(rows, acc, s):
    for t in range(4):
        pltpu.sync_copy(rows.at[0], acc.at[pl.ds(s * 632 + t * 128, 128)])
    pltpu.sync_copy(rows.at[0, pl.ds(0, 120)],
                    acc.at[pl.ds(s * 632 + 512, 120)])


def _copy_out(acc, out_hbm, c, s):
    pltpu.sync_copy(acc.at[pl.ds(s * 624, 624)],
                    out_hbm.at[c, pl.ds(s * 624, 624)])

    @pl.when(s == 15)
    def _():
        pltpu.sync_copy(acc.at[pl.ds(9984, 16)],
                        out_hbm.at[c, pl.ds(9984, 16)])


def _scale_rows(rows, b, wbuf, slot, wrow):
    def grp(g, carry):
        w16 = wbuf[slot, wrow, pl.ds(g * 16, 16)]
        for i in range(16):
            r = g * 16 + i
            wb = jnp.broadcast_to(w16[i], (16,))
            for q in range(8):
                sl = pl.ds(q * 16, 16)
                rows[b, r, sl] = rows[b, r, sl] * wb
        return carry

    lax.fori_loop(0, 8, grp, 0)


def _edge_loop(x_hbm, stage_idx_fn, src_idx, dst_idx, rows, acc,
               sem_g, base, n_chunks, ig, scale_fn):
    def g_slot(k):
        return ((k // ig) % 2, k % ig)

    stage_idx_fn(0)
    for b in range(NB):
        pltpu.async_copy(x_hbm.at[src_idx.at[g_slot(b)]], rows.at[b],
                         sem_g.at[b])

    def outer(jo, carry):
        for b in range(NB):
            k = jo * NB + b
            slot, row = g_slot(k)
            pltpu.make_async_copy(
                x_hbm.at[src_idx.at[slot, row]], rows.at[b],
                sem_g.at[b]).wait()
            scale_fn(rows, b, slot, row)
            pltpu.sync_copy(rows.at[b], acc.at[dst_idx.at[slot, row]],
                            add=True)

            @pl.when(jnp.logical_and((k + 2) % ig == 0, k + 2 < n_chunks))
            def _():
                stage_idx_fn((k + 2) // ig)

            @pl.when(k + 2 < n_chunks)
            def _():
                slot2, row2 = g_slot(k + 2)
                pltpu.async_copy(x_hbm.at[src_idx.at[slot2, row2]],
                                 rows.at[b], sem_g.at[b])
        return carry

    lax.fori_loop(0, n_chunks // NB, outer, 0)


IG1 = 16
IG23 = 32


@functools.partial(
    pl.kernel,
    out_type=jax.ShapeDtypeStruct((2, N, D), jnp.float32),
    mesh=plsc.VectorSubcoreMesh(**_MESH),
    compiler_params=pltpu.CompilerParams(needs_layout_passes=False),
    scratch_types=[
        pltpu.VMEM((2, IG1, CW), jnp.int32),
        pltpu.VMEM((2, IG1, CW), jnp.int32),
        pltpu.VMEM((2, IG1, CW), jnp.float32),
        pltpu.VMEM((NB, CW, D), jnp.float32),
        pltpu.VMEM_SHARED((ACC_ROWS, D), jnp.float32),
        pltpu.SemaphoreType.DMA((NB,)),
    ],
)
def _sc_conv1(x_hbm, eix_hbm, w_hbm, out_hbm,
              src_idx, dst_idx, wbuf, rows, acc, sem_g):
    c = lax.axis_index("c")
    s = lax.axis_index("s")
    base = (c * 16 + s) * 80
    n_chunks = 80

    _zero_buf(rows)
    _zero_acc_stripe(rows, acc, s)
    plsc.subcore_barrier()

    def stage_idx_fn(g):
        rb = base + g * IG1
        slot = g % 2
        pltpu.sync_copy(eix_hbm.at[0, pl.ds(rb, IG1)], src_idx.at[slot])
        pltpu.sync_copy(eix_hbm.at[1, pl.ds(rb, IG1)], dst_idx.at[slot])
        pltpu.sync_copy(w_hbm.at[pl.ds(rb, IG1)], wbuf.at[slot])

    def scale_fn(rows_, b, slot, row):
        _scale_rows(rows_, b, wbuf, slot, row)

    _edge_loop(x_hbm, stage_idx_fn, src_idx, dst_idx, rows, acc,
               sem_g, base, n_chunks, IG1, scale_fn)

    plsc.subcore_barrier()
    _copy_out(acc, out_hbm, c, s)


@functools.partial(
    pl.kernel,
    out_type=jax.ShapeDtypeStruct((2, N, D), jnp.float32),
    mesh=plsc.VectorSubcoreMesh(**_MESH),
    compiler_params=pltpu.CompilerParams(needs_layout_passes=False),
    scratch_types=[
        pltpu.VMEM((2, IG23, CW), jnp.int32),
        pltpu.VMEM((2, IG23, CW), jnp.int32),
        pltpu.VMEM((NB, CW, D), jnp.float32),
        pltpu.VMEM_SHARED((ACC_ROWS, D), jnp.float32),
        pltpu.SemaphoreType.DMA((NB,)),
    ],
)
def _sc_conv23(x2_hbm, x3_hbm, eix_hbm, out_hbm,
               src_idx, dst_idx, rows, acc, sem_g):
    c = lax.axis_index("c")
    s = lax.axis_index("s")
    base = s * 160
    n_chunks = 160

    _zero_buf(rows)
    _zero_acc_stripe(rows, acc, s)
    plsc.subcore_barrier()

    def stage_idx_fn(g):
        rb = base + g * IG23
        slot = g % 2
        pltpu.sync_copy(eix_hbm.at[0, pl.ds(rb, IG23)], src_idx.at[slot])
        pltpu.sync_copy(eix_hbm.at[1, pl.ds(rb, IG23)], dst_idx.at[slot])

    noscale = lambda rows_, b, slot, row: None

    @pl.when(c == 0)
    def _():
        _edge_loop(x2_hbm, stage_idx_fn, src_idx, dst_idx, rows,
                   acc, sem_g, base, n_chunks, IG23, noscale)

    @pl.when(c == 1)
    def _():
        _edge_loop(x3_hbm, stage_idx_fn, src_idx, dst_idx, rows,
                   acc, sem_g, base, n_chunks, IG23, noscale)

    plsc.subcore_barrier()
    _copy_out(acc, out_hbm, c, s)


def _tc_conv1_combine(p1, x_meas, W_rel1, b_rel1, W_root1):
    BR = 2000
    grid = (N // BR,)

    def body(p1_ref, xm_ref, wr1_ref, b1_ref, wo1_ref, mov_ref):
        f32 = jnp.float32
        a1 = p1_ref[0] + p1_ref[1]
        m = (jnp.dot(a1, wr1_ref[...], preferred_element_type=f32)
             + b1_ref[...]
             + jnp.dot(xm_ref[...], wo1_ref[...], preferred_element_type=f32))
        mov_ref[...] = jnp.maximum(m, 0.0)

    full = lambda shape: pl.BlockSpec(shape, lambda i: (0,) * len(shape))
    return pl.pallas_call(
        body,
        grid=grid,
        in_specs=[
            pl.BlockSpec((2, BR, D), lambda i: (0, i, 0)),
            pl.BlockSpec((BR, D), lambda i: (i, 0)),
            full((D, D)), full((1, D)), full((D, D)),
        ],
        out_specs=pl.BlockSpec((BR, D), lambda i: (i, 0)),
        out_shape=jax.ShapeDtypeStruct((N, D), jnp.float32),
    )(p1, x_meas, W_rel1, b_rel1.reshape(1, D), W_root1)


def _tc_final(agg23, x_dem, W_rel2, b_rel2, W_root2,
              W_rel3, b_rel3, W_root3, W_lin, b_lin):
    BR = 2000
    grid = (N // BR,)

    def body(agg_ref, xd_ref, wr2_ref, b2_ref, wo2_ref,
             wr3_ref, b3_ref, wo3_ref, wl_ref, bl_ref, out_ref):
        f32 = jnp.float32
        a2 = agg_ref[0]
        a3 = agg_ref[1]
        u1 = (jnp.dot(a2, wr2_ref[...], preferred_element_type=f32)
              + b2_ref[...]
              + jnp.dot(xd_ref[...], wo2_ref[...], preferred_element_type=f32))
        u1 = jnp.maximum(u1, 0.0)
        u = (jnp.dot(a3, wr3_ref[...], preferred_element_type=f32)
             + b3_ref[...]
             + jnp.dot(u1, wo3_ref[...], preferred_element_type=f32))
        u = jnp.maximum(u, 0.0)
        out_ref[...] = (jnp.dot(u, wl_ref[...], preferred_element_type=f32)
                        + bl_ref[...])

    full = lambda shape: pl.BlockSpec(shape, lambda i: (0,) * len(shape))
    return pl.pallas_call(
        body,
        grid=grid,
        in_specs=[
            pl.BlockSpec((2, BR, D), lambda i: (0, i, 0)),
            pl.BlockSpec((BR, D), lambda i: (i, 0)),
            full((D, D)), full((1, D)), full((D, D)),
            full((D, D)), full((1, D)), full((D, D)),
            full((D, O)), full((1, O)),
        ],
        out_specs=pl.BlockSpec((BR, O), lambda i: (i, 0)),
        out_shape=jax.ShapeDtypeStruct((N, O), jnp.float32),
    )(agg23, x_dem, W_rel2, b_rel2.reshape(1, D), W_root2,
      W_rel3, b_rel3.reshape(1, D), W_root3, W_lin, b_lin.reshape(1, O))


def _pad_eix(eix):
    pe = EROWS * CW - E
    ar = jnp.arange(pe, dtype=jnp.int32)
    pad = jnp.stack([ar % N, N + ar % (ACC_ROWS - N)])
    return jnp.concatenate([eix, pad], axis=1).reshape(2, EROWS, CW)


def kernel(x_measurement, x_demand, edge_index_mp, edge_index_rev,
           edge_weight, W_rel1, b_rel1, W_root1, W_rel2, b_rel2, W_root2,
           W_rel3, b_rel3, W_root3, W_lin, b_lin):
    eix_mp = _pad_eix(edge_index_mp)
    eix_rv = _pad_eix(edge_index_rev)
    w_mp = jnp.pad(edge_weight, (0, EROWS * CW - E)).reshape(EROWS, CW)

    p1 = _sc_conv1(x_measurement, eix_mp, w_mp)
    movie_x = _tc_conv1_combine(p1, x_measurement, W_rel1, b_rel1, W_root1)
    agg23 = _sc_conv23(x_measurement, movie_x, eix_rv)
    return _tc_final(agg23, x_demand, W_rel2, b_rel2, W_root2,
                     W_rel3, b_rel3, W_root3, W_lin, b_lin)


# speedup vs baseline: 1.0068x; 1.0005x over previous
import functools

import jax
import jax.numpy as jnp
from jax import lax
from jax.experimental import pallas as pl
from jax.experimental.pallas import tpu as pltpu
from jax.experimental.pallas import tpu_sc as plsc

N = 10000
E = 320000
D = 128
O = 64
ACC_ROWS = 10112
EROWS = 2560
CW = 128
NB = 2

_MESH = dict(core_axis_name="c", subcore_axis_name="s", num_cores=2,
             num_subcores=16)


def _zero_buf(rows):
    z = jnp.zeros((16,), jnp.float32)

    def body(r, carry):
        for q in range(8):
            rows[0, r, pl.ds(q * 16, 16)] = z
        return carry

    lax.fori_loop(0, 128, body, 0)


def _zero_acc_stripe(rows, acc, s):
    for t in range(4):
        pltpu.sync_copy(rows.at[0], acc.at[pl.ds(s * 632 + t * 128, 128)])
    pltpu.sync_copy(rows.at[0, pl.ds(0, 120)],
                    acc.at[pl.ds(s * 632 + 512, 120)])


def _copy_out(acc, out_hbm, c, s):
    pltpu.sync_copy(acc.at[pl.ds(s * 624, 624)],
                    out_hbm.at[c, pl.ds(s * 624, 624)])

    @pl.when(s == 15)
    def _():
        pltpu.sync_copy(acc.at[pl.ds(9984, 16)],
                        out_hbm.at[c, pl.ds(9984, 16)])


def _scale_rows(rows, b, wbuf, slot, wrow):
    def grp(g, carry):
        w16 = wbuf[slot, wrow, pl.ds(g * 16, 16)]
        for i in range(16):
            r = g * 16 + i
            wb = jnp.broadcast_to(w16[i], (16,))
            for q in range(8):
                sl = pl.ds(q * 16, 16)
                rows[b, r, sl] = rows[b, r, sl] * wb
        return carry

    lax.fori_loop(0, 8, grp, 0)


def _edge_loop(x_hbm, stage_idx_fn, src_idx, dst_idx, rows, acc,
               sem_g, base, n_chunks, ig, scale_fn):
    def g_slot(k):
        return ((k // ig) % 2, k % ig)

    stage_idx_fn(0)
    for b in range(NB):
        pltpu.async_copy(x_hbm.at[src_idx.at[g_slot(b)]], rows.at[b],
                         sem_g.at[b], priority=1)

    def outer(jo, carry):
        for b in range(NB):
            k = jo * NB + b
            slot, row = g_slot(k)
            pltpu.make_async_copy(
                x_hbm.at[src_idx.at[slot, row]], rows.at[b],
                sem_g.at[b]).wait()
            scale_fn(rows, b, slot, row)
            pltpu.sync_copy(rows.at[b], acc.at[dst_idx.at[slot, row]],
                            add=True)

            @pl.when(jnp.logical_and((k + 2) % ig == 0, k + 2 < n_chunks))
            def _():
                stage_idx_fn((k + 2) // ig)

            @pl.when(k + 2 < n_chunks)
            def _():
                slot2, row2 = g_slot(k + 2)
                pltpu.async_copy(x_hbm.at[src_idx.at[slot2, row2]],
                                 rows.at[b], sem_g.at[b], priority=1)
        return carry

    lax.fori_loop(0, n_chunks // NB, outer, 0)


IG1 = 16
IG23 = 32


@functools.partial(
    pl.kernel,
    out_type=jax.ShapeDtypeStruct((2, N, D), jnp.float32),
    mesh=plsc.VectorSubcoreMesh(**_MESH),
    compiler_params=pltpu.CompilerParams(needs_layout_passes=False),
    scratch_types=[
        pltpu.VMEM((2, IG1, CW), jnp.int32),
        pltpu.VMEM((2, IG1, CW), jnp.int32),
        pltpu.VMEM((2, IG1, CW), jnp.float32),
        pltpu.VMEM((NB, CW, D), jnp.float32),
        pltpu.VMEM_SHARED((ACC_ROWS, D), jnp.float32),
        pltpu.SemaphoreType.DMA((NB,)),
    ],
)
def _sc_conv1(x_hbm, eix_hbm, w_hbm, out_hbm,
              src_idx, dst_idx, wbuf, rows, acc, sem_g):
    c = lax.axis_index("c")
    s = lax.axis_index("s")
    base = (c * 16 + s) * 80
    n_chunks = 80

    _zero_buf(rows)
    _zero_acc_stripe(rows, acc, s)
    plsc.subcore_barrier()

    def stage_idx_fn(g):
        rb = base + g * IG1
        slot = g % 2
        pltpu.sync_copy(eix_hbm.at[0, pl.ds(rb, IG1)], src_idx.at[slot])
        pltpu.sync_copy(eix_hbm.at[1, pl.ds(rb, IG1)], dst_idx.at[slot])
        pltpu.sync_copy(w_hbm.at[pl.ds(rb, IG1)], wbuf.at[slot])

    def scale_fn(rows_, b, slot, row):
        _scale_rows(rows_, b, wbuf, slot, row)

    _edge_loop(x_hbm, stage_idx_fn, src_idx, dst_idx, rows, acc,
               sem_g, base, n_chunks, IG1, scale_fn)

    plsc.subcore_barrier()
    _copy_out(acc, out_hbm, c, s)


@functools.partial(
    pl.kernel,
    out_type=jax.ShapeDtypeStruct((2, N, D), jnp.float32),
    mesh=plsc.VectorSubcoreMesh(**_MESH),
    compiler_params=pltpu.CompilerParams(needs_layout_passes=False),
    scratch_types=[
        pltpu.VMEM((2, IG23, CW), jnp.int32),
        pltpu.VMEM((2, IG23, CW), jnp.int32),
        pltpu.VMEM((NB, CW, D), jnp.float32),
        pltpu.VMEM_SHARED((ACC_ROWS, D), jnp.float32),
        pltpu.SemaphoreType.DMA((NB,)),
    ],
)
def _sc_conv23(x2_hbm, x3_hbm, eix_hbm, out_hbm,
               src_idx, dst_idx, rows, acc, sem_g):
    c = lax.axis_index("c")
    s = lax.axis_index("s")
    base = s * 160
    n_chunks = 160

    _zero_buf(rows)
    _zero_acc_stripe(rows, acc, s)
    plsc.subcore_barrier()

    def stage_idx_fn(g):
        rb = base + g * IG23
        slot = g % 2
        pltpu.sync_copy(eix_hbm.at[0, pl.ds(rb, IG23)], src_idx.at[slot])
        pltpu.sync_copy(eix_hbm.at[1, pl.ds(rb, IG23)], dst_idx.at[slot])

    noscale = lambda rows_, b, slot, row: None

    @pl.when(c == 0)
    def _():
        _edge_loop(x2_hbm, stage_idx_fn, src_idx, dst_idx, rows,
                   acc, sem_g, base, n_chunks, IG23, noscale)

    @pl.when(c == 1)
    def _():
        _edge_loop(x3_hbm, stage_idx_fn, src_idx, dst_idx, rows,
                   acc, sem_g, base, n_chunks, IG23, noscale)

    plsc.subcore_barrier()
    _copy_out(acc, out_hbm, c, s)


def _tc_conv1_combine(p1, x_meas, W_rel1, b_rel1, W_root1):
    BR = 2000
    grid = (N // BR,)

    def body(p1_ref, xm_ref, wr1_ref, b1_ref, wo1_ref, mov_ref):
        f32 = jnp.float32
        a1 = p1_ref[0] + p1_ref[1]
        m = (jnp.dot(a1, wr1_ref[...], preferred_element_type=f32)
             + b1_ref[...]
             + jnp.dot(xm_ref[...], wo1_ref[...], preferred_element_type=f32))
        mov_ref[...] = jnp.maximum(m, 0.0)

    full = lambda shape: pl.BlockSpec(shape, lambda i: (0,) * len(shape))
    return pl.pallas_call(
        body,
        grid=grid,
        in_specs=[
            pl.BlockSpec((2, BR, D), lambda i: (0, i, 0)),
            pl.BlockSpec((BR, D), lambda i: (i, 0)),
            full((D, D)), full((1, D)), full((D, D)),
        ],
        out_specs=pl.BlockSpec((BR, D), lambda i: (i, 0)),
        out_shape=jax.ShapeDtypeStruct((N, D), jnp.float32),
    )(p1, x_meas, W_rel1, b_rel1.reshape(1, D), W_root1)


def _tc_final(agg23, x_dem, W_rel2, b_rel2, W_root2,
              W_rel3, b_rel3, W_root3, W_lin, b_lin):
    BR = 2000
    grid = (N // BR,)

    def body(agg_ref, xd_ref, wr2_ref, b2_ref, wo2_ref,
             wr3_ref, b3_ref, wo3_ref, wl_ref, bl_ref, out_ref):
        f32 = jnp.float32
        a2 = agg_ref[0]
        a3 = agg_ref[1]
        u1 = (jnp.dot(a2, wr2_ref[...], preferred_element_type=f32)
              + b2_ref[...]
              + jnp.dot(xd_ref[...], wo2_ref[...], preferred_element_type=f32))
        u1 = jnp.maximum(u1, 0.0)
        u = (jnp.dot(a3, wr3_ref[...], preferred_element_type=f32)
             + b3_ref[...]
             + jnp.dot(u1, wo3_ref[...], preferred_element_type=f32))
        u = jnp.maximum(u, 0.0)
        out_ref[...] = (jnp.dot(u, wl_ref[...], preferred_element_type=f32)
                        + bl_ref[...])

    full = lambda shape: pl.BlockSpec(shape, lambda i: (0,) * len(shape))
    return pl.pallas_call(
        body,
        grid=grid,
        in_specs=[
            pl.BlockSpec((2, BR, D), lambda i: (0, i, 0)),
            pl.BlockSpec((BR, D), lambda i: (i, 0)),
            full((D, D)), full((1, D)), full((D, D)),
            full((D, D)), full((1, D)), full((D, D)),
            full((D, O)), full((1, O)),
        ],
        out_specs=pl.BlockSpec((BR, O), lambda i: (i, 0)),
        out_shape=jax.ShapeDtypeStruct((N, O), jnp.float32),
    )(agg23, x_dem, W_rel2, b_rel2.reshape(1, D), W_root2,
      W_rel3, b_rel3.reshape(1, D), W_root3, W_lin, b_lin.reshape(1, O))


def _pad_eix(eix):
    pe = EROWS * CW - E
    ar = jnp.arange(pe, dtype=jnp.int32)
    pad = jnp.stack([ar % N, N + ar % (ACC_ROWS - N)])
    return jnp.concatenate([eix, pad], axis=1).reshape(2, EROWS, CW)


def kernel(x_measurement, x_demand, edge_index_mp, edge_index_rev,
           edge_weight, W_rel1, b_rel1, W_root1, W_rel2, b_rel2, W_root2,
           W_rel3, b_rel3, W_root3, W_lin, b_lin):
    eix_mp = _pad_eix(edge_index_mp)
    eix_rv = _pad_eix(edge_index_rev)
    w_mp = jnp.pad(edge_weight, (0, EROWS * CW - E)).reshape(EROWS, CW)

    p1 = _sc_conv1(x_measurement, eix_mp, w_mp)
    movie_x = _tc_conv1_combine(p1, x_measurement, W_rel1, b_rel1, W_root1)
    agg23 = _sc_conv23(x_measurement, movie_x, eix_rv)
    return _tc_final(agg23, x_demand, W_rel2, b_rel2, W_root2,
                     W_rel3, b_rel3, W_root3, W_lin, b_lin)
